# baseline (speedup 1.0000x reference)
import functools

import jax
import jax.numpy as jnp
from jax import lax
from jax.experimental import pallas as pl
from jax.experimental.pallas import tpu as pltpu
from jax.experimental.pallas import tpu_sc as plsc

_N = 10000
_NP = 10240
_E = 320000
_D = 128
_NW = 32
_PER_W = _E // _NW
_CHUNK = 128
_NCHUNK = _PER_W // _CHUNK
_TAIL = _PER_W - _NCHUNK * _CHUNK
_DEPTH = 16
_STRIPE = _NP // 16

_sc_mesh = plsc.VectorSubcoreMesh(core_axis_name="c", subcore_axis_name="s")


@functools.partial(
    pl.kernel,
    mesh=_sc_mesh,
    out_type=(
        jax.ShapeDtypeStruct((_NP,), jnp.float32),
        jax.ShapeDtypeStruct((_NP,), jnp.float32),
    ),
    scratch_types=[
        pltpu.VMEM((_NCHUNK, _CHUNK), jnp.int32),
        pltpu.VMEM((1, _TAIL), jnp.int32),
        pltpu.VMEM((_PER_W,), jnp.float32),
        pltpu.VMEM((_STRIPE,), jnp.float32),
        pltpu.VMEM_SHARED((_NP,), jnp.float32),
        pltpu.SemaphoreType.DMA,
        pltpu.SemaphoreType.DMA,
    ],
)
def _segment_sum_sc(col_hbm, ew_hbm, out0, out1, idx_v, idx_t, val_v, zbuf,
                    acc, ld_sem, st_sem):
    c = lax.axis_index("c")
    s = lax.axis_index("s")
    wid = c * 16 + s
    base = wid * _PER_W

    pltpu.async_copy(ew_hbm.at[pl.ds(base, _PER_W)], val_v, ld_sem)
    pltpu.async_copy(col_hbm.at[pl.ds(base + _NCHUNK * _CHUNK, _TAIL)],
                     idx_t.at[0], ld_sem)

    def ld(j, carry):
        pltpu.async_copy(col_hbm.at[pl.ds(base + j * _CHUNK, _CHUNK)],
                         idx_v.at[j], ld_sem)
        return carry

    lax.fori_loop(0, _NCHUNK, ld, 0)

    for j in range(_STRIPE // 16):
        zbuf[pl.ds(j * 16, 16)] = jnp.zeros((16,), jnp.float32)
    pltpu.sync_copy(zbuf, acc.at[pl.ds(s * _STRIPE, _STRIPE)])

    pltpu.make_async_copy(ew_hbm.at[pl.ds(base, _PER_W)], val_v, ld_sem).wait()
    pltpu.make_async_copy(col_hbm.at[pl.ds(base, _TAIL)], idx_t.at[0],
                          ld_sem).wait()

    def ld_drain(j, carry):
        pltpu.make_async_copy(col_hbm.at[pl.ds(base, _CHUNK)], idx_v.at[j],
                              ld_sem).wait()
        return carry

    lax.fori_loop(0, _NCHUNK, ld_drain, 0)
    plsc.subcore_barrier()

    def fire(j, carry):
        pltpu.async_copy(val_v.at[pl.ds(j * _CHUNK, _CHUNK)],
                         acc.at[idx_v.at[j]], st_sem, add=True)
        return carry

    def wait_fire(j, carry):
        k = j - _DEPTH
        pltpu.make_async_copy(val_v.at[pl.ds(k * _CHUNK, _CHUNK)],
                              acc.at[idx_v.at[k]], st_sem).wait()
        pltpu.async_copy(val_v.at[pl.ds(j * _CHUNK, _CHUNK)],
                         acc.at[idx_v.at[j]], st_sem, add=True)
        return carry

    def drain(j, carry):
        pltpu.make_async_copy(val_v.at[pl.ds(j * _CHUNK, _CHUNK)],
                              acc.at[idx_v.at[j]], st_sem).wait()
        return carry

    lax.fori_loop(0, _DEPTH, fire, 0)
    lax.fori_loop(_DEPTH, _NCHUNK, wait_fire, 0)
    pltpu.async_copy(val_v.at[pl.ds(_NCHUNK * _CHUNK, _TAIL)],
                     acc.at[idx_t.at[0]], st_sem, add=True)
    lax.fori_loop(_NCHUNK - _DEPTH, _NCHUNK, drain, 0)
    pltpu.make_async_copy(val_v.at[pl.ds(_NCHUNK * _CHUNK, _TAIL)],
                          acc.at[idx_t.at[0]], st_sem).wait()
    plsc.subcore_barrier()

    @pl.when(jnp.logical_and(s == 0, c == 0))
    def _():
        pltpu.sync_copy(acc, out0)

    @pl.when(jnp.logical_and(s == 0, c == 1))
    def _():
        pltpu.sync_copy(acc, out1)


_R = 2048


def _pan_tc_body(x_ref, p_ref, p0_ref, p1_ref, beta_ref, out_ref, score_ref):
    nblk = _R // 128
    xb = x_ref[...]
    s1row = jax.lax.dot_general(p_ref[...], xb, (((1,), (1,)), ((), ())),
                                precision=jax.lax.Precision.HIGHEST)
    z = (beta_ref[0] * jnp.squeeze(s1row, axis=0)
         + beta_ref[1] * (p0_ref[...] + p1_ref[...]))
    sc = 1.0 / (1.0 + jnp.exp(-z))
    score_ref[...] = sc
    sc2 = jnp.reshape(sc, (1, _R))
    rows = jnp.concatenate(
        [sc2[:, 128 * r:128 * (r + 1)] for r in range(nblk)], axis=0)
    sct = jnp.transpose(rows)
    for r in range(nblk):
        out_ref[pl.ds(128 * r, 128), :] = (
            xb[128 * r:128 * (r + 1), :] * sct[:, r:r + 1])


def kernel(x, row, col, edge_weight, p, beta):
    del row
    part0, part1 = _segment_sum_sc(col, edge_weight)

    out, score = pl.pallas_call(
        _pan_tc_body,
        grid=(pl.cdiv(_N, _R),),
        in_specs=[
            pl.BlockSpec((_R, _D), lambda i: (i, 0)),
            pl.BlockSpec((1, _D), lambda i: (0, 0)),
            pl.BlockSpec((_R,), lambda i: (i,)),
            pl.BlockSpec((_R,), lambda i: (i,)),
            pl.BlockSpec(memory_space=pltpu.SMEM),
        ],
        out_specs=[
            pl.BlockSpec((_R, _D), lambda i: (i, 0)),
            pl.BlockSpec((_R,), lambda i: (i,)),
        ],
        out_shape=[
            jax.ShapeDtypeStruct((_N, _D), jnp.float32),
            jax.ShapeDtypeStruct((_N,), jnp.float32),
        ],
    )(x, p.reshape(1, _D), part0, part1, beta)

    return (out, score)


# speedup vs baseline: 1.0071x; 1.0071x over previous
import functools

import jax
import jax.numpy as jnp
from jax import lax
from jax.experimental import pallas as pl
from jax.experimental.pallas import tpu as pltpu
from jax.experimental.pallas import tpu_sc as plsc

_N = 10000
_NP = 10240
_E = 320000
_D = 128
_NW = 32
_PER_W = _E // _NW
_CHUNK = 128
_NCHUNK = _PER_W // _CHUNK
_TAIL = _PER_W - _NCHUNK * _CHUNK
_DEPTH = 16
_STRIPE = _NP // 16

_sc_mesh = plsc.VectorSubcoreMesh(core_axis_name="c", subcore_axis_name="s")


@functools.partial(
    pl.kernel,
    mesh=_sc_mesh,
    out_type=(
        jax.ShapeDtypeStruct((_NP,), jnp.float32),
        jax.ShapeDtypeStruct((_NP,), jnp.float32),
    ),
    scratch_types=[
        pltpu.VMEM((_NCHUNK, _CHUNK), jnp.int32),
        pltpu.VMEM((1, _TAIL), jnp.int32),
        pltpu.VMEM((_PER_W,), jnp.float32),
        pltpu.VMEM((_STRIPE,), jnp.float32),
        pltpu.VMEM_SHARED((_NP,), jnp.float32),
        pltpu.SemaphoreType.DMA,
        pltpu.SemaphoreType.DMA,
    ],
)
def _segment_sum_sc(col_hbm, ew_hbm, out0, out1, idx_v, idx_t, val_v, zbuf,
                    acc, ld_sem, st_sem):
    c = lax.axis_index("c")
    s = lax.axis_index("s")
    wid = c * 16 + s
    base = wid * _PER_W

    pltpu.async_copy(ew_hbm.at[pl.ds(base, _PER_W)], val_v, ld_sem)
    pltpu.async_copy(col_hbm.at[pl.ds(base + _NCHUNK * _CHUNK, _TAIL)],
                     idx_t.at[0], ld_sem)

    def ld(j, carry):
        pltpu.async_copy(col_hbm.at[pl.ds(base + j * _CHUNK, _CHUNK)],
                         idx_v.at[j], ld_sem)
        return carry

    lax.fori_loop(0, _NCHUNK, ld, 0)

    for j in range(_STRIPE // 16):
        zbuf[pl.ds(j * 16, 16)] = jnp.zeros((16,), jnp.float32)
    pltpu.sync_copy(zbuf, acc.at[pl.ds(s * _STRIPE, _STRIPE)])

    pltpu.make_async_copy(ew_hbm.at[pl.ds(base, _PER_W)], val_v, ld_sem).wait()
    pltpu.make_async_copy(ew_hbm.at[pl.ds(base, _PER_W)], val_v, ld_sem).wait()
    plsc.subcore_barrier()

    def fire(j, carry):
        pltpu.async_copy(val_v.at[pl.ds(j * _CHUNK, _CHUNK)],
                         acc.at[idx_v.at[j]], st_sem, add=True)
        return carry

    lax.fori_loop(0, _NCHUNK, fire, 0)
    pltpu.async_copy(val_v.at[pl.ds(_NCHUNK * _CHUNK, _TAIL)],
                     acc.at[idx_t.at[0]], st_sem, add=True)
    pltpu.make_async_copy(ew_hbm.at[pl.ds(base, _PER_W)], val_v, st_sem).wait()
    plsc.subcore_barrier()

    @pl.when(jnp.logical_and(s == 0, c == 0))
    def _():
        pltpu.sync_copy(acc, out0)

    @pl.when(jnp.logical_and(s == 0, c == 1))
    def _():
        pltpu.sync_copy(acc, out1)


_R = 2048


def _pan_tc_body(x_ref, p_ref, p0_ref, p1_ref, beta_ref, out_ref, score_ref):
    nblk = _R // 128
    xb = x_ref[...]
    s1row = jax.lax.dot_general(p_ref[...], xb, (((1,), (1,)), ((), ())),
                                precision=jax.lax.Precision.HIGHEST)
    z = (beta_ref[0] * jnp.squeeze(s1row, axis=0)
         + beta_ref[1] * (p0_ref[...] + p1_ref[...]))
    sc = 1.0 / (1.0 + jnp.exp(-z))
    score_ref[...] = sc
    sc2 = jnp.reshape(sc, (1, _R))
    rows = jnp.concatenate(
        [sc2[:, 128 * r:128 * (r + 1)] for r in range(nblk)], axis=0)
    sct = jnp.transpose(rows)
    for r in range(nblk):
        out_ref[pl.ds(128 * r, 128), :] = (
            xb[128 * r:128 * (r + 1), :] * sct[:, r:r + 1])


def kernel(x, row, col, edge_weight, p, beta):
    del row
    part0, part1 = _segment_sum_sc(col, edge_weight)

    out, score = pl.pallas_call(
        _pan_tc_body,
        grid=(pl.cdiv(_N, _R),),
        in_specs=[
            pl.BlockSpec((_R, _D), lambda i: (i, 0)),
            pl.BlockSpec((1, _D), lambda i: (0, 0)),
            pl.BlockSpec((_R,), lambda i: (i,)),
            pl.BlockSpec((_R,), lambda i: (i,)),
            pl.BlockSpec(memory_space=pltpu.SMEM),
        ],
        out_specs=[
            pl.BlockSpec((_R, _D), lambda i: (i, 0)),
            pl.BlockSpec((_R,), lambda i: (i,)),
        ],
        out_shape=[
            jax.ShapeDtypeStruct((_N, _D), jnp.float32),
            jax.ShapeDtypeStruct((_N,), jnp.float32),
        ],
    )(x, p.reshape(1, _D), part0, part1, beta)

    return (out, score)


# speedup vs baseline: 1.0745x; 1.0670x over previous
import functools

import jax
import jax.numpy as jnp
from jax import lax
from jax.experimental import pallas as pl
from jax.experimental.pallas import tpu as pltpu
from jax.experimental.pallas import tpu_sc as plsc

_N = 10000
_NP = 10240
_E = 320000
_D = 128
_NW = 32
_PER_W = _E // _NW
_CHUNK = 128
_NCHUNK = _PER_W // _CHUNK
_TAIL = _PER_W - _NCHUNK * _CHUNK
_DEPTH = 16
_STRIPE = _NP // 16

_sc_mesh = plsc.VectorSubcoreMesh(core_axis_name="c", subcore_axis_name="s")


@functools.partial(
    pl.kernel,
    mesh=_sc_mesh,
    out_type=(
        jax.ShapeDtypeStruct((_NP,), jnp.float32),
        jax.ShapeDtypeStruct((_NP,), jnp.float32),
    ),
    scratch_types=[
        pltpu.VMEM((_NCHUNK, _CHUNK), jnp.int32),
        pltpu.VMEM((1, _TAIL), jnp.int32),
        pltpu.VMEM((_PER_W,), jnp.float32),
        pltpu.VMEM((_STRIPE,), jnp.float32),
        pltpu.VMEM_SHARED((_NP,), jnp.float32),
        pltpu.SemaphoreType.DMA,
        pltpu.SemaphoreType.DMA,
    ],
)
def _segment_sum_sc(col_hbm, ew_hbm, out0, out1, idx_v, idx_t, val_v, zbuf,
                    acc, ld_sem, st_sem):
    c = lax.axis_index("c")
    s = lax.axis_index("s")
    wid = c * 16 + s
    base = wid * _PER_W

    pltpu.async_copy(ew_hbm.at[pl.ds(base, _PER_W)], val_v, ld_sem)
    pltpu.async_copy(col_hbm.at[pl.ds(base + _NCHUNK * _CHUNK, _TAIL)],
                     idx_t.at[0], ld_sem)

    def ld(j, carry):
        pltpu.async_copy(col_hbm.at[pl.ds(base + j * _CHUNK, _CHUNK)],
                         idx_v.at[j], ld_sem)
        return carry

    lax.fori_loop(0, _NCHUNK, ld, 0)

    for j in range(_STRIPE // 16):
        zbuf[pl.ds(j * 16, 16)] = jnp.zeros((16,), jnp.float32)
    pltpu.sync_copy(zbuf, acc.at[pl.ds(s * _STRIPE, _STRIPE)])

    pltpu.make_async_copy(ew_hbm.at[pl.ds(base, _PER_W)], val_v, ld_sem).wait()
    pltpu.make_async_copy(ew_hbm.at[pl.ds(base, _PER_W)], val_v, ld_sem).wait()
    plsc.subcore_barrier()

    def fire(j, carry):
        pltpu.async_copy(val_v.at[pl.ds(j * _CHUNK, _CHUNK)],
                         acc.at[idx_v.at[j]], st_sem, add=True)
        return carry

    lax.fori_loop(0, _NCHUNK, fire, 0)
    pltpu.async_copy(val_v.at[pl.ds(_NCHUNK * _CHUNK, _TAIL)],
                     acc.at[idx_t.at[0]], st_sem, add=True)
    pltpu.make_async_copy(ew_hbm.at[pl.ds(base, _PER_W)], val_v, st_sem).wait()
    plsc.subcore_barrier()

    @pl.when(jnp.logical_and(s == 0, c == 0))
    def _():
        pltpu.sync_copy(acc, out0)

    @pl.when(jnp.logical_and(s == 0, c == 1))
    def _():
        pltpu.sync_copy(acc, out1)


_R = 2048


def _mxu_t(a, n):
    eye = jnp.float32(1.0) * (
        jax.lax.broadcasted_iota(jnp.int32, (n, n), 0)
        == jax.lax.broadcasted_iota(jnp.int32, (n, n), 1))
    return jax.lax.dot_general(a, eye, (((0,), (0,)), ((), ())),
                               precision=jax.lax.Precision.HIGHEST)


def _pan_tc_body(x_ref, p_ref, p0_ref, p1_ref, beta_ref, out_ref, score_ref):
    nblk = _R // 128
    xb = x_ref[...]
    s1t = jnp.concatenate(
        [jnp.sum(xb[128 * r:128 * (r + 1), :] * p_ref[...], axis=1,
                 keepdims=True) for r in range(nblk)], axis=1)
    ppt = _mxu_t(p0_ref[...] + p1_ref[...], nblk)
    z = beta_ref[0] * s1t + beta_ref[1] * ppt
    sct = 1.0 / (1.0 + jnp.exp(-z))
    for r in range(nblk):
        out_ref[pl.ds(128 * r, 128), :] = (
            xb[128 * r:128 * (r + 1), :] * sct[:, r:r + 1])
    sc_lane = _mxu_t(sct, 128)
    score_ref[...] = jnp.squeeze(
        jnp.concatenate([sc_lane[r:r + 1, :] for r in range(nblk)], axis=1),
        axis=0)


def kernel(x, row, col, edge_weight, p, beta):
    del row
    part0, part1 = _segment_sum_sc(col, edge_weight)

    out, score = pl.pallas_call(
        _pan_tc_body,
        grid=(pl.cdiv(_N, _R),),
        in_specs=[
            pl.BlockSpec((_R, _D), lambda i: (i, 0)),
            pl.BlockSpec((1, _D), lambda i: (0, 0)),
            pl.BlockSpec((_R // 128, 128), lambda i: (i, 0)),
            pl.BlockSpec((_R // 128, 128), lambda i: (i, 0)),
            pl.BlockSpec(memory_space=pltpu.SMEM),
        ],
        out_specs=[
            pl.BlockSpec((_R, _D), lambda i: (i, 0)),
            pl.BlockSpec((_R,), lambda i: (i,)),
        ],
        out_shape=[
            jax.ShapeDtypeStruct((_N, _D), jnp.float32),
            jax.ShapeDtypeStruct((_N,), jnp.float32),
        ],
    )(x, p.reshape(1, _D), part0.reshape(_NP // 128, 128),
      part1.reshape(_NP // 128, 128), beta)

    return (out, score)
